# baseline (speedup 1.0000x reference)
import functools

import jax
import jax.numpy as jnp
from jax import lax
from jax.experimental import pallas as pl
from jax.experimental.pallas import tpu as pltpu
from jax.experimental.pallas import tpu_sc as plsc

VOCAB = 1000
BATCH = 1024
SEQ = 50
SCH = 16
DIM = VOCAB
DIMP = 1024
ROWSP = VOCAB + 104
MAIN = 896
TAIL = DIM - MAIN

_INFO = plsc.get_sparse_core_info()
NC = _INFO.num_cores
NS = _INFO.num_subcores
NW = NC * NS
BPW = BATCH // NW


def _tail_rows(tail_v, buf, nrows):
  def tail_row(r, carry):
    for i in range(TAIL // 16):
      tail_v[r, pl.ds(i * 16, 16)] = buf[r, pl.ds(MAIN + i * 16, 16)]
    tail_v[r, pl.ds(TAIL - 16, 16)] = buf[r, pl.ds(MAIN + TAIL - 16, 16)]
    return carry

  lax.fori_loop(0, nrows, tail_row, 0)


def _make_sc_gather16():
  mesh = plsc.VectorSubcoreMesh(core_axis_name="c", subcore_axis_name="s")
  NCH = BPW // 2

  @functools.partial(
      pl.kernel,
      mesh=mesh,
      out_type=jax.ShapeDtypeStruct((BATCH, SCH, DIM), jnp.float32),
      scratch_types=[
          pltpu.VMEM((NCH, 2 * SCH), jnp.int32),
          pltpu.VMEM((2 * SCH, DIMP), jnp.float32),
          pltpu.VMEM((2 * SCH, DIMP), jnp.float32),
          pltpu.VMEM((2 * SCH, TAIL), jnp.float32),
          pltpu.SemaphoreType.DMA,
          pltpu.SemaphoreType.DMA,
      ],
      compiler_params=pltpu.CompilerParams(use_tc_tiling_on_sc=True),
  )
  def body(table_hbm, idx_hbm, out_hbm, idx_v, buf0, buf1, tail_v,
           sem0, sem1):
    wid = lax.axis_index("s") * NC + lax.axis_index("c")
    base = wid * BPW

    pltpu.sync_copy(idx_hbm.at[wid], idx_v)

    def gather(c, buf, sem):
      return pltpu.make_async_copy(table_hbm.at[idx_v.at[c]], buf, sem)

    def writeback(c, buf):
      _tail_rows(tail_v, buf, 2 * SCH)
      for k in range(2):
        b = base + 2 * c + k
        pltpu.sync_copy(buf.at[pl.ds(k * SCH, SCH), pl.ds(0, MAIN)],
                        out_hbm.at[b, :, pl.ds(0, MAIN)])
        pltpu.sync_copy(tail_v.at[pl.ds(k * SCH, SCH), :],
                        out_hbm.at[b, :, pl.ds(MAIN, TAIL)])

    gather(0, buf0, sem0).start()
    gather(1, buf1, sem1).start()

    def step(i, carry):
      c0 = 2 * i
      c1 = c0 + 1
      gather(c0, buf0, sem0).wait()
      writeback(c0, buf0)

      @pl.when(c0 + 2 < NCH)
      def _():
        gather(c0 + 2, buf0, sem0).start()

      gather(c1, buf1, sem1).wait()
      writeback(c1, buf1)

      @pl.when(c1 + 2 < NCH)
      def _():
        gather(c1 + 2, buf1, sem1).start()

      return carry

    lax.fori_loop(0, NCH // 2, step, 0)

  return body


def _make_sc_gather2():
  mesh = plsc.VectorSubcoreMesh(core_axis_name="c", subcore_axis_name="s")

  @functools.partial(
      pl.kernel,
      mesh=mesh,
      out_type=jax.ShapeDtypeStruct((BATCH, 2, DIM), jnp.float32),
      scratch_types=[
          pltpu.VMEM((BPW, 2), jnp.int32),
          pltpu.VMEM((2, DIMP), jnp.float32),
          pltpu.VMEM((2, DIMP), jnp.float32),
          pltpu.VMEM((2, TAIL), jnp.float32),
          pltpu.SemaphoreType.DMA,
          pltpu.SemaphoreType.DMA,
      ],
      compiler_params=pltpu.CompilerParams(use_tc_tiling_on_sc=True),
  )
  def body(table_hbm, idx_hbm, out_hbm, idx_v, buf0, buf1, tail_v,
           sem0, sem1):
    wid = lax.axis_index("s") * NC + lax.axis_index("c")
    base = wid * BPW

    pltpu.sync_copy(idx_hbm.at[wid], idx_v)

    def gather(c, buf, sem):
      return pltpu.make_async_copy(table_hbm.at[idx_v.at[c]], buf, sem)

    def writeback(c, buf):
      _tail_rows(tail_v, buf, 2)
      pltpu.sync_copy(buf.at[:, pl.ds(0, MAIN)],
                      out_hbm.at[base + c, :, pl.ds(0, MAIN)])
      pltpu.sync_copy(tail_v, out_hbm.at[base + c, :, pl.ds(MAIN, TAIL)])

    gather(0, buf0, sem0).start()
    gather(1, buf1, sem1).start()

    def step(i, carry):
      c0 = 2 * i
      c1 = c0 + 1
      gather(c0, buf0, sem0).wait()
      writeback(c0, buf0)

      @pl.when(c0 + 2 < BPW)
      def _():
        gather(c0 + 2, buf0, sem0).start()

      gather(c1, buf1, sem1).wait()
      writeback(c1, buf1)

      @pl.when(c1 + 2 < BPW)
      def _():
        gather(c1 + 2, buf1, sem1).start()

      return carry

    lax.fori_loop(0, BPW // 2, step, 0)

  return body


_sc_gather16 = _make_sc_gather16()
_sc_gather2 = _make_sc_gather2()


def kernel(idx, token_embedding_table):
  idx_w = idx.astype(jnp.int32)
  table_p = jnp.pad(token_embedding_table,
                    ((0, ROWSP - VOCAB), (0, DIMP - DIM)))
  parts = []
  for p in range(SEQ // SCH):
    idx_p = idx_w[:, p * SCH:(p + 1) * SCH].reshape(NW, BPW // 2, 2 * SCH)
    parts.append(_sc_gather16(table_p, idx_p))
  idx_t = idx_w[:, SEQ - 2:].reshape(NW, BPW, 2)
  parts.append(_sc_gather2(table_p, idx_t))
  return jnp.concatenate(parts, axis=1)
